# baseline (speedup 1.0000x reference)
import functools

import jax
import jax.numpy as jnp
from jax import lax
from jax.experimental import pallas as pl
from jax.experimental.pallas import tpu as pltpu
from jax.experimental.pallas import tpu_sc as plsc

BATCH = 16384
FIELDS = 26
EMBED_DIM = 16
VOCAB = 1000000

NUM_CORES = 2
NUM_SUBCORES = 16
NUM_WORKERS = NUM_CORES * NUM_SUBCORES
TOTAL = BATCH * FIELDS
PER_WORKER = TOTAL // NUM_WORKERS
CHUNK = 1664
NUM_CHUNKS = PER_WORKER // CHUNK
LANES = 16

VT_FULL = 7812
B_VT = 6
UNITS_PER_TABLE = VT_FULL // B_VT
UNITS = 2 * UNITS_PER_TABLE
UNIT_V = B_VT * 128
UNIT_ROWS = UNIT_V * EMBED_DIM // 128
S_ROWS = VOCAB * EMBED_DIM // 128

_mesh = plsc.VectorSubcoreMesh(
    core_axis_name="c", subcore_axis_name="s",
    num_cores=NUM_CORES, num_subcores=NUM_SUBCORES)


@functools.partial(
    pl.kernel,
    out_type=(jax.ShapeDtypeStruct((S_ROWS, 128), jnp.float32),
              jax.ShapeDtypeStruct((S_ROWS, 128), jnp.float32)),
    mesh=_mesh,
    scratch_types=[
        pltpu.VMEM((EMBED_DIM, UNIT_V), jnp.float32),
        pltpu.VMEM((UNIT_ROWS, 128), jnp.float32),
        pltpu.VMEM((8, 128), jnp.float32),
    ],
    compiler_params=pltpu.CompilerParams(needs_layout_passes=False),
)
def _convert(t0_hbm, t1_hbm, tail0_hbm, tail1_hbm, s0_hbm, s1_hbm,
             tbuf, rbuf, tailbuf):
    wid = lax.axis_index("s") * NUM_CORES + lax.axis_index("c")
    q, r = UNITS // NUM_WORKERS, UNITS % NUM_WORKERS
    lo = wid * q + jnp.minimum(wid, r)
    cnt = q + jnp.where(wid < r, 1, 0)
    d_iota = lax.iota(jnp.int32, LANES)

    def unit_body(u, carry):
        in_t1 = u >= UNITS_PER_TABLE
        blk = jnp.where(in_t1, u - UNITS_PER_TABLE, u)
        v0 = blk * UNIT_V

        @pl.when(jnp.logical_not(in_t1))
        def _():
            pltpu.sync_copy(t0_hbm.at[:, pl.ds(v0, UNIT_V)], tbuf)

        @pl.when(in_t1)
        def _():
            pltpu.sync_copy(t1_hbm.at[:, pl.ds(v0, UNIT_V)], tbuf)

        def row_body(rr, c):
            for g in range(8):
                v_loc = rr * 8 + g
                row = plsc.load_gather(
                    tbuf, [d_iota, jnp.broadcast_to(v_loc, (LANES,))])
                rbuf[rr, pl.ds(g * LANES, LANES)] = row
            return c

        lax.fori_loop(0, UNIT_ROWS, row_body, 0)
        ro = blk * UNIT_ROWS

        @pl.when(jnp.logical_not(in_t1))
        def _():
            pltpu.sync_copy(rbuf, s0_hbm.at[pl.ds(ro, UNIT_ROWS), :])

        @pl.when(in_t1)
        def _():
            pltpu.sync_copy(rbuf, s1_hbm.at[pl.ds(ro, UNIT_ROWS), :])

        return carry

    lax.fori_loop(lo, lo + cnt, unit_body, 0)

    @pl.when(wid == NUM_WORKERS - 1)
    def _():
        pltpu.sync_copy(tail0_hbm, tailbuf)
        pltpu.sync_copy(tailbuf, s0_hbm.at[pl.ds(S_ROWS - 8, 8), :])
        pltpu.sync_copy(tail1_hbm, tailbuf)
        pltpu.sync_copy(tailbuf, s1_hbm.at[pl.ds(S_ROWS - 8, 8), :])


@functools.partial(
    pl.kernel,
    out_type=jax.ShapeDtypeStruct((2 * TOTAL, EMBED_DIM), jnp.float32),
    mesh=_mesh,
    scratch_types=[
        pltpu.VMEM((CHUNK,), jnp.int32),
        pltpu.VMEM((CHUNK, EMBED_DIM), jnp.float32),
        pltpu.VMEM((CHUNK, EMBED_DIM), jnp.float32),
        pltpu.VMEM((CHUNK,), jnp.int32),
        pltpu.VMEM((CHUNK,), jnp.int32),
        pltpu.SemaphoreType.DMA,
        pltpu.SemaphoreType.DMA,
        pltpu.SemaphoreType.DMA,
        pltpu.SemaphoreType.DMA,
    ],
    compiler_params=pltpu.CompilerParams(use_tc_tiling_on_sc=False),
)
def _stack_embed(x_hbm, t0_hbm, t1_hbm, out_hbm,
                 xidx_v, r0_v, r1_v, o0_v, o1_v, gs0, gs1, ss0, ss1):
    wid = lax.axis_index("s") * NUM_CORES + lax.axis_index("c")
    base = wid * PER_WORKER

    def body(i, carry):
        off = pl.multiple_of(base + i * CHUNK, 8)
        pltpu.sync_copy(x_hbm.at[pl.ds(off, CHUNK)], xidx_v)
        cp0 = pltpu.async_copy(t0_hbm.at[xidx_v], r0_v, gs0)
        cp1 = pltpu.async_copy(t1_hbm.at[xidx_v], r1_v, gs1)

        two_iota = 2 * lax.iota(jnp.int32, LANES)

        def gen(j, c):
            v0 = (2 * off + 2 * LANES * j) + two_iota
            o0_v[pl.ds(j * LANES, LANES)] = v0
            o1_v[pl.ds(j * LANES, LANES)] = v0 + 1
            return c

        lax.fori_loop(0, CHUNK // LANES, gen, 0)

        cp0.wait()
        sc0 = pltpu.async_copy(r0_v, out_hbm.at[o0_v], ss0)
        cp1.wait()
        sc1 = pltpu.async_copy(r1_v, out_hbm.at[o1_v], ss1)
        sc0.wait()
        sc1.wait()
        return carry

    lax.fori_loop(0, NUM_CHUNKS, body, 0)


def kernel(x, table0, table1):
    t0t = table0.T
    t1t = table1.T
    tail0 = lax.slice(table0, (VT_FULL * 128, 0), (VOCAB, EMBED_DIM))
    tail1 = lax.slice(table1, (VT_FULL * 128, 0), (VOCAB, EMBED_DIM))
    s0, s1 = _convert(t0t, t1t, tail0.reshape(8, 128), tail1.reshape(8, 128))
    out = _stack_embed(x.reshape(TOTAL),
                       s0.reshape(VOCAB, EMBED_DIM),
                       s1.reshape(VOCAB, EMBED_DIM))
    return out.reshape(BATCH, FIELDS, 2 * EMBED_DIM)


# speedup vs baseline: 1.6203x; 1.6203x over previous
import functools

import jax
import jax.numpy as jnp
from jax import lax
from jax.experimental import pallas as pl
from jax.experimental.pallas import tpu as pltpu
from jax.experimental.pallas import tpu_sc as plsc

BATCH = 16384
FIELDS = 26
EMBED_DIM = 16
VOCAB = 1000000

NUM_CORES = 2
NUM_SUBCORES = 16
NUM_WORKERS = NUM_CORES * NUM_SUBCORES
TOTAL = BATCH * FIELDS
PER_WORKER = TOTAL // NUM_WORKERS
CHUNK = 1664
NUM_CHUNKS = PER_WORKER // CHUNK
LANES = 16

VT_FULL = 7812
B_VT = 4
UNITS_PER_TABLE = VT_FULL // B_VT
UNITS = 2 * UNITS_PER_TABLE
UNIT_V = B_VT * 128
UNIT_ROWS = UNIT_V * EMBED_DIM // 128
S_ROWS = VOCAB * EMBED_DIM // 128

_mesh = plsc.VectorSubcoreMesh(
    core_axis_name="c", subcore_axis_name="s",
    num_cores=NUM_CORES, num_subcores=NUM_SUBCORES)


UNIT_W = UNIT_V * EMBED_DIM
VGRP = UNIT_V // LANES


@functools.partial(
    pl.kernel,
    out_type=(jax.ShapeDtypeStruct((VOCAB * EMBED_DIM,), jnp.float32),
              jax.ShapeDtypeStruct((VOCAB * EMBED_DIM,), jnp.float32)),
    mesh=_mesh,
    scratch_types=[
        pltpu.VMEM((EMBED_DIM, UNIT_V), jnp.float32),
        pltpu.VMEM((EMBED_DIM, UNIT_V), jnp.float32),
        pltpu.VMEM((UNIT_W,), jnp.float32),
        pltpu.VMEM((UNIT_W,), jnp.float32),
        pltpu.VMEM((1024,), jnp.float32),
        pltpu.SemaphoreType.DMA,
        pltpu.SemaphoreType.DMA,
        pltpu.SemaphoreType.DMA,
        pltpu.SemaphoreType.DMA,
    ],
    compiler_params=pltpu.CompilerParams(needs_layout_passes=False,
                                         disable_bounds_checks=True),
)
def _convert(t0_hbm, t1_hbm, tail0_hbm, tail1_hbm, s0_hbm, s1_hbm,
             tbufa, tbufb, rbufa, rbufb, tailbuf, in0, in1, out0, out1):
    wid = lax.axis_index("s") * NUM_CORES + lax.axis_index("c")
    q, r = UNITS // NUM_WORKERS, UNITS % NUM_WORKERS
    lo = wid * q + jnp.minimum(wid, r)
    cnt = q + jnp.where(wid < r, 1, 0)
    iota16 = lax.iota(jnp.int32, LANES) * EMBED_DIM
    in_sems = (in0, in1)
    out_sems = (out0, out1)
    tbufs = (tbufa, tbufb)
    rbufs = (rbufa, rbufb)

    def start_in(u, slot):
        in_t1 = u >= UNITS_PER_TABLE
        v0 = jnp.where(in_t1, u - UNITS_PER_TABLE, u) * UNIT_V

        @pl.when(jnp.logical_not(in_t1))
        def _():
            pltpu.async_copy(t0_hbm.at[:, pl.ds(v0, UNIT_V)],
                             tbufs[slot], in_sems[slot])

        @pl.when(in_t1)
        def _():
            pltpu.async_copy(t1_hbm.at[:, pl.ds(v0, UNIT_V)],
                             tbufs[slot], in_sems[slot])

    def wait_in(slot):
        pltpu.make_async_copy(t0_hbm.at[:, pl.ds(0, UNIT_V)],
                              tbufs[slot], in_sems[slot]).wait()

    def start_out(u, slot):
        in_t1 = u >= UNITS_PER_TABLE
        wo = jnp.where(in_t1, u - UNITS_PER_TABLE, u) * UNIT_W

        @pl.when(jnp.logical_not(in_t1))
        def _():
            pltpu.async_copy(rbufs[slot],
                             s0_hbm.at[pl.ds(wo, UNIT_W)],
                             out_sems[slot])

        @pl.when(in_t1)
        def _():
            pltpu.async_copy(rbufs[slot],
                             s1_hbm.at[pl.ds(wo, UNIT_W)],
                             out_sems[slot])

    def wait_out(slot):
        pltpu.make_async_copy(rbufs[slot],
                              s0_hbm.at[pl.ds(0, UNIT_W)],
                              out_sems[slot]).wait()

    def transpose_unit(slot):
        for d in range(EMBED_DIM):
            base = iota16 + d
            for vg in range(VGRP):
                vals = tbufs[slot][d, pl.ds(vg * LANES, LANES)]
                plsc.store_scatter(rbufs[slot],
                                   [base + vg * LANES * EMBED_DIM], vals)

    for k in range(2):
        @pl.when(k < cnt)
        def _():
            start_in(lo + k, k)

    def unit_body(i, carry):
        u = lo + i
        islot = i % 2
        for slot in range(2):
            @pl.when(islot == slot)
            def _():
                wait_in(slot)

                @pl.when(i >= 2)
                def _():
                    wait_out(slot)

                transpose_unit(slot)
                start_out(u, slot)

                @pl.when(i + 2 < cnt)
                def _():
                    start_in(u + 2, slot)
        return carry

    lax.fori_loop(0, cnt, unit_body, 0)
    for slot in range(2):
        @pl.when((cnt >= 1) & ((cnt - 1) % 2 == slot))
        def _():
            wait_out(slot)

        @pl.when((cnt >= 2) & ((cnt - 2) % 2 == slot))
        def _():
            wait_out(slot)

    @pl.when(wid == NUM_WORKERS - 1)
    def _():
        pltpu.sync_copy(tail0_hbm, tailbuf)
        pltpu.sync_copy(tailbuf, s0_hbm.at[pl.ds(VT_FULL * 128 * EMBED_DIM,
                                                 1024)])
        pltpu.sync_copy(tail1_hbm, tailbuf)
        pltpu.sync_copy(tailbuf, s1_hbm.at[pl.ds(VT_FULL * 128 * EMBED_DIM,
                                                 1024)])


@functools.partial(
    pl.kernel,
    out_type=jax.ShapeDtypeStruct((2 * TOTAL, EMBED_DIM), jnp.float32),
    mesh=_mesh,
    scratch_types=[
        pltpu.VMEM((CHUNK,), jnp.int32),
        pltpu.VMEM((CHUNK, EMBED_DIM), jnp.float32),
        pltpu.VMEM((CHUNK, EMBED_DIM), jnp.float32),
        pltpu.VMEM((CHUNK,), jnp.int32),
        pltpu.VMEM((CHUNK,), jnp.int32),
        pltpu.SemaphoreType.DMA,
        pltpu.SemaphoreType.DMA,
        pltpu.SemaphoreType.DMA,
        pltpu.SemaphoreType.DMA,
    ],
    compiler_params=pltpu.CompilerParams(use_tc_tiling_on_sc=False),
)
def _stack_embed(x_hbm, t0_hbm, t1_hbm, out_hbm,
                 xidx_v, r0_v, r1_v, o0_v, o1_v, gs0, gs1, ss0, ss1):
    wid = lax.axis_index("s") * NUM_CORES + lax.axis_index("c")
    base = wid * PER_WORKER

    def body(i, carry):
        off = pl.multiple_of(base + i * CHUNK, 8)
        pltpu.sync_copy(x_hbm.at[pl.ds(off, CHUNK)], xidx_v)
        cp0 = pltpu.async_copy(t0_hbm.at[xidx_v], r0_v, gs0)
        cp1 = pltpu.async_copy(t1_hbm.at[xidx_v], r1_v, gs1)

        two_iota = 2 * lax.iota(jnp.int32, LANES)

        def gen(j, c):
            v0 = (2 * off + 2 * LANES * j) + two_iota
            o0_v[pl.ds(j * LANES, LANES)] = v0
            o1_v[pl.ds(j * LANES, LANES)] = v0 + 1
            return c

        lax.fori_loop(0, CHUNK // LANES, gen, 0)

        cp0.wait()
        sc0 = pltpu.async_copy(r0_v, out_hbm.at[o0_v], ss0)
        cp1.wait()
        sc1 = pltpu.async_copy(r1_v, out_hbm.at[o1_v], ss1)
        sc0.wait()
        sc1.wait()
        return carry

    lax.fori_loop(0, NUM_CHUNKS, body, 0)


def kernel(x, table0, table1):
    t0t = table0.T
    t1t = table1.T
    tail0 = lax.slice(table0, (VT_FULL * 128, 0), (VOCAB, EMBED_DIM))
    tail1 = lax.slice(table1, (VT_FULL * 128, 0), (VOCAB, EMBED_DIM))
    s0, s1 = _convert(t0t, t1t, tail0.reshape(1024), tail1.reshape(1024))
    out = _stack_embed(x.reshape(TOTAL),
                       s0.reshape(VOCAB, EMBED_DIM),
                       s1.reshape(VOCAB, EMBED_DIM))
    return out.reshape(BATCH, FIELDS, 2 * EMBED_DIM)


# speedup vs baseline: 2.5312x; 1.5622x over previous
import functools

import jax
import jax.numpy as jnp
from jax import lax
from jax.experimental import pallas as pl
from jax.experimental.pallas import tpu as pltpu
from jax.experimental.pallas import tpu_sc as plsc

BATCH = 16384
FIELDS = 26
EMBED_DIM = 16
VOCAB = 1000000

NUM_CORES = 2
NUM_SUBCORES = 16
NUM_WORKERS = NUM_CORES * NUM_SUBCORES
TOTAL = BATCH * FIELDS
PER_WORKER = TOTAL // NUM_WORKERS
CHUNK = 1664
NUM_CHUNKS = PER_WORKER // CHUNK
LANES = 16

VT_FULL = 7812
B_VT = 4
UNITS_PER_TABLE = VT_FULL // B_VT
UNITS = 2 * UNITS_PER_TABLE
UNIT_V = B_VT * 128
UNIT_ROWS = UNIT_V * EMBED_DIM // 128
S_ROWS = VOCAB * EMBED_DIM // 128

_mesh = plsc.VectorSubcoreMesh(
    core_axis_name="c", subcore_axis_name="s",
    num_cores=NUM_CORES, num_subcores=NUM_SUBCORES)


UNIT_W = UNIT_V * EMBED_DIM
VGRP = UNIT_V // LANES


@functools.partial(
    pl.kernel,
    out_type=(jax.ShapeDtypeStruct((VOCAB * EMBED_DIM,), jnp.float32),
              jax.ShapeDtypeStruct((VOCAB * EMBED_DIM,), jnp.float32)),
    mesh=_mesh,
    scratch_types=[
        pltpu.VMEM((EMBED_DIM, UNIT_V), jnp.float32),
        pltpu.VMEM((EMBED_DIM, UNIT_V), jnp.float32),
        pltpu.VMEM((UNIT_W,), jnp.float32),
        pltpu.VMEM((UNIT_W,), jnp.float32),
        pltpu.VMEM((1024,), jnp.float32),
        pltpu.SemaphoreType.DMA,
        pltpu.SemaphoreType.DMA,
        pltpu.SemaphoreType.DMA,
        pltpu.SemaphoreType.DMA,
    ],
    compiler_params=pltpu.CompilerParams(needs_layout_passes=False,
                                         disable_bounds_checks=True),
)
def _convert(t0_hbm, t1_hbm, tail0_hbm, tail1_hbm, s0_hbm, s1_hbm,
             tbufa, tbufb, rbufa, rbufb, tailbuf, in0, in1, out0, out1):
    wid = lax.axis_index("s") * NUM_CORES + lax.axis_index("c")
    q, r = UNITS // NUM_WORKERS, UNITS % NUM_WORKERS
    lo = wid * q + jnp.minimum(wid, r)
    cnt = q + jnp.where(wid < r, 1, 0)
    iota16 = lax.iota(jnp.int32, LANES) * EMBED_DIM
    in_sems = (in0, in1)
    out_sems = (out0, out1)
    tbufs = (tbufa, tbufb)
    rbufs = (rbufa, rbufb)

    def start_in(u, slot):
        in_t1 = u >= UNITS_PER_TABLE
        v0 = jnp.where(in_t1, u - UNITS_PER_TABLE, u) * UNIT_V

        @pl.when(jnp.logical_not(in_t1))
        def _():
            pltpu.async_copy(t0_hbm.at[:, pl.ds(v0, UNIT_V)],
                             tbufs[slot], in_sems[slot])

        @pl.when(in_t1)
        def _():
            pltpu.async_copy(t1_hbm.at[:, pl.ds(v0, UNIT_V)],
                             tbufs[slot], in_sems[slot])

    def wait_in(slot):
        pltpu.make_async_copy(t0_hbm.at[:, pl.ds(0, UNIT_V)],
                              tbufs[slot], in_sems[slot]).wait()

    def start_out(u, slot):
        in_t1 = u >= UNITS_PER_TABLE
        wo = jnp.where(in_t1, u - UNITS_PER_TABLE, u) * UNIT_W

        @pl.when(jnp.logical_not(in_t1))
        def _():
            pltpu.async_copy(rbufs[slot],
                             s0_hbm.at[pl.ds(wo, UNIT_W)],
                             out_sems[slot])

        @pl.when(in_t1)
        def _():
            pltpu.async_copy(rbufs[slot],
                             s1_hbm.at[pl.ds(wo, UNIT_W)],
                             out_sems[slot])

    def wait_out(slot):
        pltpu.make_async_copy(rbufs[slot],
                              s0_hbm.at[pl.ds(0, UNIT_W)],
                              out_sems[slot]).wait()

    bases = [iota16 + d for d in range(EMBED_DIM)]

    def transpose_unit(slot):
        @plsc.parallel_loop(0, VGRP, unroll=4)
        def _(vg):
            col = vg * (LANES * EMBED_DIM)
            for d in range(EMBED_DIM):
                vals = tbufs[slot][d, pl.ds(vg * LANES, LANES)]
                plsc.store_scatter(rbufs[slot], [bases[d] + col], vals)

    for k in range(2):
        @pl.when(k < cnt)
        def _():
            start_in(lo + k, k)

    def unit_body(i, carry):
        u = lo + i
        islot = i % 2
        for slot in range(2):
            @pl.when(islot == slot)
            def _():
                wait_in(slot)

                @pl.when(i >= 2)
                def _():
                    wait_out(slot)

                transpose_unit(slot)
                start_out(u, slot)

                @pl.when(i + 2 < cnt)
                def _():
                    start_in(u + 2, slot)
        return carry

    lax.fori_loop(0, cnt, unit_body, 0)
    for slot in range(2):
        @pl.when((cnt >= 1) & ((cnt - 1) % 2 == slot))
        def _():
            wait_out(slot)

        @pl.when((cnt >= 2) & ((cnt - 2) % 2 == slot))
        def _():
            wait_out(slot)

    @pl.when(wid == NUM_WORKERS - 1)
    def _():
        pltpu.sync_copy(tail0_hbm, tailbuf)
        pltpu.sync_copy(tailbuf, s0_hbm.at[pl.ds(VT_FULL * 128 * EMBED_DIM,
                                                 1024)])
        pltpu.sync_copy(tail1_hbm, tailbuf)
        pltpu.sync_copy(tailbuf, s1_hbm.at[pl.ds(VT_FULL * 128 * EMBED_DIM,
                                                 1024)])


@functools.partial(
    pl.kernel,
    out_type=jax.ShapeDtypeStruct((2 * TOTAL, EMBED_DIM), jnp.float32),
    mesh=_mesh,
    scratch_types=[
        pltpu.VMEM((CHUNK,), jnp.int32),
        pltpu.VMEM((CHUNK, EMBED_DIM), jnp.float32),
        pltpu.VMEM((CHUNK, EMBED_DIM), jnp.float32),
        pltpu.VMEM((CHUNK,), jnp.int32),
        pltpu.VMEM((CHUNK,), jnp.int32),
        pltpu.SemaphoreType.DMA,
        pltpu.SemaphoreType.DMA,
        pltpu.SemaphoreType.DMA,
        pltpu.SemaphoreType.DMA,
    ],
    compiler_params=pltpu.CompilerParams(use_tc_tiling_on_sc=False),
)
def _stack_embed(x_hbm, t0_hbm, t1_hbm, out_hbm,
                 xidx_v, r0_v, r1_v, o0_v, o1_v, gs0, gs1, ss0, ss1):
    wid = lax.axis_index("s") * NUM_CORES + lax.axis_index("c")
    base = wid * PER_WORKER

    def body(i, carry):
        off = pl.multiple_of(base + i * CHUNK, 8)
        pltpu.sync_copy(x_hbm.at[pl.ds(off, CHUNK)], xidx_v)
        cp0 = pltpu.async_copy(t0_hbm.at[xidx_v], r0_v, gs0)
        cp1 = pltpu.async_copy(t1_hbm.at[xidx_v], r1_v, gs1)

        two_iota = 2 * lax.iota(jnp.int32, LANES)

        def gen(j, c):
            v0 = (2 * off + 2 * LANES * j) + two_iota
            o0_v[pl.ds(j * LANES, LANES)] = v0
            o1_v[pl.ds(j * LANES, LANES)] = v0 + 1
            return c

        lax.fori_loop(0, CHUNK // LANES, gen, 0)

        cp0.wait()
        sc0 = pltpu.async_copy(r0_v, out_hbm.at[o0_v], ss0)
        cp1.wait()
        sc1 = pltpu.async_copy(r1_v, out_hbm.at[o1_v], ss1)
        sc0.wait()
        sc1.wait()
        return carry

    lax.fori_loop(0, NUM_CHUNKS, body, 0)


def kernel(x, table0, table1):
    t0t = table0.T
    t1t = table1.T
    tail0 = lax.slice(table0, (VT_FULL * 128, 0), (VOCAB, EMBED_DIM))
    tail1 = lax.slice(table1, (VT_FULL * 128, 0), (VOCAB, EMBED_DIM))
    s0, s1 = _convert(t0t, t1t, tail0.reshape(1024), tail1.reshape(1024))
    out = _stack_embed(x.reshape(TOTAL),
                       s0.reshape(VOCAB, EMBED_DIM),
                       s1.reshape(VOCAB, EMBED_DIM))
    return out.reshape(BATCH, FIELDS, 2 * EMBED_DIM)


# speedup vs baseline: 4.4276x; 1.7492x over previous
import functools

import jax
import jax.numpy as jnp
from jax import lax
from jax.experimental import pallas as pl
from jax.experimental.pallas import tpu as pltpu
from jax.experimental.pallas import tpu_sc as plsc

BATCH = 16384
FIELDS = 26
EMBED_DIM = 16
VOCAB = 1000000

NUM_CORES = 2
NUM_SUBCORES = 16
NUM_WORKERS = NUM_CORES * NUM_SUBCORES
TOTAL = BATCH * FIELDS
PER_WORKER = TOTAL // NUM_WORKERS
CHUNK = 1664
NUM_CHUNKS = PER_WORKER // CHUNK
LANES = 16

VT_FULL = 7812
B_VT = 4
UNITS_PER_TABLE = VT_FULL // B_VT
UNITS = 2 * UNITS_PER_TABLE
UNIT_V = B_VT * 128
UNIT_ROWS = UNIT_V * EMBED_DIM // 128
S_ROWS = VOCAB * EMBED_DIM // 128

_mesh = plsc.VectorSubcoreMesh(
    core_axis_name="c", subcore_axis_name="s",
    num_cores=NUM_CORES, num_subcores=NUM_SUBCORES)


UNIT_W = UNIT_V * EMBED_DIM
VGRP = UNIT_V // LANES


@functools.partial(
    pl.kernel,
    out_type=(jax.ShapeDtypeStruct((VOCAB * EMBED_DIM,), jnp.float32),
              jax.ShapeDtypeStruct((VOCAB * EMBED_DIM,), jnp.float32)),
    mesh=_mesh,
    scratch_types=[
        pltpu.VMEM((EMBED_DIM, UNIT_V), jnp.float32),
        pltpu.VMEM((EMBED_DIM, UNIT_V), jnp.float32),
        pltpu.VMEM((UNIT_W,), jnp.float32),
        pltpu.VMEM((UNIT_W,), jnp.float32),
        pltpu.VMEM((1024,), jnp.float32),
        pltpu.SemaphoreType.DMA,
        pltpu.SemaphoreType.DMA,
        pltpu.SemaphoreType.DMA,
        pltpu.SemaphoreType.DMA,
    ],
    compiler_params=pltpu.CompilerParams(needs_layout_passes=False,
                                         disable_bounds_checks=True),
)
def _convert(t0_hbm, t1_hbm, tail0_hbm, tail1_hbm, s0_hbm, s1_hbm,
             tbufa, tbufb, rbufa, rbufb, tailbuf, in0, in1, out0, out1):
    wid = lax.axis_index("s") * NUM_CORES + lax.axis_index("c")
    q, r = UNITS // NUM_WORKERS, UNITS % NUM_WORKERS
    lo = wid * q + jnp.minimum(wid, r)
    cnt = q + jnp.where(wid < r, 1, 0)
    iota16 = lax.iota(jnp.int32, LANES) * EMBED_DIM
    in_sems = (in0, in1)
    out_sems = (out0, out1)
    tbufs = (tbufa, tbufb)
    rbufs = (rbufa, rbufb)

    def start_in(u, slot):
        in_t1 = u >= UNITS_PER_TABLE
        v0 = jnp.where(in_t1, u - UNITS_PER_TABLE, u) * UNIT_V

        @pl.when(jnp.logical_not(in_t1))
        def _():
            pltpu.async_copy(t0_hbm.at[:, pl.ds(v0, UNIT_V)],
                             tbufs[slot], in_sems[slot])

        @pl.when(in_t1)
        def _():
            pltpu.async_copy(t1_hbm.at[:, pl.ds(v0, UNIT_V)],
                             tbufs[slot], in_sems[slot])

    def wait_in(slot):
        pltpu.make_async_copy(t0_hbm.at[:, pl.ds(0, UNIT_V)],
                              tbufs[slot], in_sems[slot]).wait()

    def start_out(u, slot):
        in_t1 = u >= UNITS_PER_TABLE
        wo = jnp.where(in_t1, u - UNITS_PER_TABLE, u) * UNIT_W

        @pl.when(jnp.logical_not(in_t1))
        def _():
            pltpu.async_copy(rbufs[slot],
                             s0_hbm.at[pl.ds(wo, UNIT_W)],
                             out_sems[slot])

        @pl.when(in_t1)
        def _():
            pltpu.async_copy(rbufs[slot],
                             s1_hbm.at[pl.ds(wo, UNIT_W)],
                             out_sems[slot])

    def wait_out(slot):
        pltpu.make_async_copy(rbufs[slot],
                              s0_hbm.at[pl.ds(0, UNIT_W)],
                              out_sems[slot]).wait()

    bases = [iota16 + d for d in range(EMBED_DIM)]

    def transpose_unit(slot):
        @plsc.parallel_loop(0, VGRP, unroll=4)
        def _(vg):
            col = vg * (LANES * EMBED_DIM)
            for d in range(EMBED_DIM):
                vals = tbufs[slot][d, pl.ds(vg * LANES, LANES)]
                plsc.store_scatter(rbufs[slot], [bases[d] + col], vals)

    for k in range(2):
        @pl.when(k < cnt)
        def _():
            start_in(lo + k, k)

    def unit_body(i, carry):
        u = lo + i
        islot = i % 2
        for slot in range(2):
            @pl.when(islot == slot)
            def _():
                wait_in(slot)

                @pl.when(i >= 2)
                def _():
                    wait_out(slot)

                transpose_unit(slot)
                start_out(u, slot)

                @pl.when(i + 2 < cnt)
                def _():
                    start_in(u + 2, slot)
        return carry

    lax.fori_loop(0, cnt, unit_body, 0)
    for slot in range(2):
        @pl.when((cnt >= 1) & ((cnt - 1) % 2 == slot))
        def _():
            wait_out(slot)

        @pl.when((cnt >= 2) & ((cnt - 2) % 2 == slot))
        def _():
            wait_out(slot)

    @pl.when(wid == NUM_WORKERS - 1)
    def _():
        pltpu.sync_copy(tail0_hbm, tailbuf)
        pltpu.sync_copy(tailbuf, s0_hbm.at[pl.ds(VT_FULL * 128 * EMBED_DIM,
                                                 1024)])
        pltpu.sync_copy(tail1_hbm, tailbuf)
        pltpu.sync_copy(tailbuf, s1_hbm.at[pl.ds(VT_FULL * 128 * EMBED_DIM,
                                                 1024)])


BT_UNITS = BATCH // 128
BT_PER_W = BT_UNITS // NUM_WORKERS
XIN = 128 * FIELDS
OUT_WORDS = BATCH * FIELDS * 2 * EMBED_DIM
FBLK = 4 * 8 * 128


@functools.partial(
    pl.kernel,
    out_type=jax.ShapeDtypeStruct((OUT_WORDS,), jnp.float32),
    mesh=_mesh,
    scratch_types=[
        pltpu.VMEM((XIN,), jnp.int32),
        pltpu.VMEM((XIN,), jnp.int32),
        pltpu.VMEM((128, EMBED_DIM), jnp.float32),
        pltpu.VMEM((128, EMBED_DIM), jnp.float32),
        pltpu.VMEM((128, EMBED_DIM), jnp.float32),
        pltpu.VMEM((128, EMBED_DIM), jnp.float32),
        pltpu.VMEM((FBLK,), jnp.float32),
        pltpu.VMEM((FBLK,), jnp.float32),
        pltpu.SemaphoreType.DMA,
        pltpu.SemaphoreType.DMA,
        pltpu.SemaphoreType.DMA,
        pltpu.SemaphoreType.DMA,
        pltpu.SemaphoreType.DMA,
        pltpu.SemaphoreType.DMA,
    ],
    compiler_params=pltpu.CompilerParams(use_tc_tiling_on_sc=False,
                                         needs_layout_passes=False,
                                         disable_bounds_checks=True),
)
def _stack_embed(x_hbm, t0_hbm, t1_hbm, out_hbm,
                 xin, xcols, r0a, r0b, r1a, r1b, obla, oblb,
                 g0a, g0b, g1a, g1b, oa, ob):
    wid = lax.axis_index("s") * NUM_CORES + lax.axis_index("c")
    iota = lax.iota(jnp.int32, LANES)
    colbase = [iota * FIELDS + 416 * g for g in range(8)]
    bidx = [iota + LANES * g for g in range(8)]
    r0s, r1s = (r0a, r0b), (r1a, r1b)
    obls = (obla, oblb)
    gsem0, gsem1 = (g0a, g0b), (g1a, g1b)
    osems = (oa, ob)

    def start_gather(f, slot):
        idxr = xcols.at[pl.ds(f * 128, 128)]
        pltpu.async_copy(t0_hbm.at[idxr], r0s[slot], gsem0[slot])
        pltpu.async_copy(t1_hbm.at[idxr], r1s[slot], gsem1[slot])

    def wait_gather(slot):
        pltpu.make_async_copy(t0_hbm.at[pl.ds(0, 128), :],
                              r0s[slot], gsem0[slot]).wait()
        pltpu.make_async_copy(t1_hbm.at[pl.ds(0, 128), :],
                              r1s[slot], gsem1[slot]).wait()

    def start_outs(f, bt, slot):
        for dt in range(4):
            off = ((f * 4 + dt) * BT_UNITS + bt) * 1024
            pltpu.async_copy(obls[slot].at[pl.ds(dt * 1024, 1024)],
                             out_hbm.at[pl.ds(off, 1024)], osems[slot])

    def wait_outs(slot):
        pltpu.make_async_copy(obls[slot], out_hbm.at[pl.ds(0, FBLK)],
                              osems[slot]).wait()

    def assemble(slot):
        @plsc.parallel_loop(0, EMBED_DIM, unroll=2)
        def _(d):
            dv = jnp.broadcast_to(d, (LANES,))
            for g in range(8):
                v0 = plsc.load_gather(r0s[slot], [bidx[g], dv])
                obls[slot][pl.ds(d * 128 + g * LANES, LANES)] = v0
                v1 = plsc.load_gather(r1s[slot], [bidx[g], dv])
                obls[slot][pl.ds((EMBED_DIM + d) * 128 + g * LANES,
                                 LANES)] = v1

    def bt_body(j, carry):
        bt = wid * BT_PER_W + j
        pltpu.sync_copy(x_hbm.at[pl.ds(bt * XIN, XIN)], xin)

        @plsc.parallel_loop(0, FIELDS, unroll=2)
        def _(f):
            for g in range(8):
                vals = plsc.load_gather(xin, [colbase[g] + f])
                xcols[pl.ds(f * 128 + g * LANES, LANES)] = vals

        for k in range(2):
            start_gather(k, k)

        def f_body(f, c):
            for slot in range(2):
                @pl.when(f % 2 == slot)
                def _():
                    wait_gather(slot)

                    @pl.when(f >= 2)
                    def _():
                        wait_outs(slot)

                    assemble(slot)
                    start_outs(f, bt, slot)

                    @pl.when(f + 2 < FIELDS)
                    def _():
                        start_gather(f + 2, slot)
            return c

        lax.fori_loop(0, FIELDS, f_body, 0)
        for slot in range(2):
            wait_outs(slot)
        return carry

    lax.fori_loop(0, BT_PER_W, bt_body, 0)


def kernel(x, table0, table1):
    t0t = table0.T
    t1t = table1.T
    tail0 = lax.slice(table0, (VT_FULL * 128, 0), (VOCAB, EMBED_DIM))
    tail1 = lax.slice(table1, (VT_FULL * 128, 0), (VOCAB, EMBED_DIM))
    s0, s1 = _convert(t0t, t1t, tail0.reshape(1024), tail1.reshape(1024))
    out = _stack_embed(x.reshape(TOTAL),
                       s0.reshape(VOCAB, EMBED_DIM),
                       s1.reshape(VOCAB, EMBED_DIM))
    return (out.reshape(FIELDS, 4, BT_UNITS, 8, 128)
            .transpose(2, 4, 0, 1, 3)
            .reshape(BATCH, FIELDS, 2 * EMBED_DIM))


# speedup vs baseline: 4.6580x; 1.0520x over previous
import functools

import jax
import jax.numpy as jnp
from jax import lax
from jax.experimental import pallas as pl
from jax.experimental.pallas import tpu as pltpu
from jax.experimental.pallas import tpu_sc as plsc

BATCH = 16384
FIELDS = 26
EMBED_DIM = 16
VOCAB = 1000000

NUM_CORES = 2
NUM_SUBCORES = 16
NUM_WORKERS = NUM_CORES * NUM_SUBCORES
TOTAL = BATCH * FIELDS
PER_WORKER = TOTAL // NUM_WORKERS
CHUNK = 1664
NUM_CHUNKS = PER_WORKER // CHUNK
LANES = 16

VT_FULL = 7812
B_VT = 6
UNITS_PER_TABLE = VT_FULL // B_VT
UNITS = 2 * UNITS_PER_TABLE
UNIT_V = B_VT * 128
UNIT_ROWS = UNIT_V * EMBED_DIM // 128
S_ROWS = VOCAB * EMBED_DIM // 128

_mesh = plsc.VectorSubcoreMesh(
    core_axis_name="c", subcore_axis_name="s",
    num_cores=NUM_CORES, num_subcores=NUM_SUBCORES)


UNIT_W = UNIT_V * EMBED_DIM
VGRP = UNIT_V // LANES


@functools.partial(
    pl.kernel,
    out_type=(jax.ShapeDtypeStruct((VOCAB * EMBED_DIM,), jnp.float32),
              jax.ShapeDtypeStruct((VOCAB * EMBED_DIM,), jnp.float32)),
    mesh=_mesh,
    scratch_types=[
        pltpu.VMEM((EMBED_DIM, UNIT_V), jnp.float32),
        pltpu.VMEM((EMBED_DIM, UNIT_V), jnp.float32),
        pltpu.VMEM((UNIT_W,), jnp.float32),
        pltpu.VMEM((UNIT_W,), jnp.float32),
        pltpu.VMEM((1024,), jnp.float32),
        pltpu.SemaphoreType.DMA,
        pltpu.SemaphoreType.DMA,
        pltpu.SemaphoreType.DMA,
        pltpu.SemaphoreType.DMA,
    ],
    compiler_params=pltpu.CompilerParams(needs_layout_passes=False,
                                         disable_bounds_checks=True),
)
def _convert(t0_hbm, t1_hbm, tail0_hbm, tail1_hbm, s0_hbm, s1_hbm,
             tbufa, tbufb, rbufa, rbufb, tailbuf, in0, in1, out0, out1):
    wid = lax.axis_index("s") * NUM_CORES + lax.axis_index("c")
    q, r = UNITS // NUM_WORKERS, UNITS % NUM_WORKERS
    lo = wid * q + jnp.minimum(wid, r)
    cnt = q + jnp.where(wid < r, 1, 0)
    iota16 = lax.iota(jnp.int32, LANES) * EMBED_DIM
    in_sems = (in0, in1)
    out_sems = (out0, out1)
    tbufs = (tbufa, tbufb)
    rbufs = (rbufa, rbufb)

    def start_in(u, slot):
        in_t1 = u >= UNITS_PER_TABLE
        v0 = jnp.where(in_t1, u - UNITS_PER_TABLE, u) * UNIT_V

        @pl.when(jnp.logical_not(in_t1))
        def _():
            pltpu.async_copy(t0_hbm.at[:, pl.ds(v0, UNIT_V)],
                             tbufs[slot], in_sems[slot])

        @pl.when(in_t1)
        def _():
            pltpu.async_copy(t1_hbm.at[:, pl.ds(v0, UNIT_V)],
                             tbufs[slot], in_sems[slot])

    def wait_in(slot):
        pltpu.make_async_copy(t0_hbm.at[:, pl.ds(0, UNIT_V)],
                              tbufs[slot], in_sems[slot]).wait()

    def start_out(u, slot):
        in_t1 = u >= UNITS_PER_TABLE
        wo = jnp.where(in_t1, u - UNITS_PER_TABLE, u) * UNIT_W

        @pl.when(jnp.logical_not(in_t1))
        def _():
            pltpu.async_copy(rbufs[slot],
                             s0_hbm.at[pl.ds(wo, UNIT_W)],
                             out_sems[slot])

        @pl.when(in_t1)
        def _():
            pltpu.async_copy(rbufs[slot],
                             s1_hbm.at[pl.ds(wo, UNIT_W)],
                             out_sems[slot])

    def wait_out(slot):
        pltpu.make_async_copy(rbufs[slot],
                              s0_hbm.at[pl.ds(0, UNIT_W)],
                              out_sems[slot]).wait()

    bases = [iota16 + d for d in range(EMBED_DIM)]

    def transpose_unit(slot):
        @plsc.parallel_loop(0, VGRP, unroll=4)
        def _(vg):
            col = vg * (LANES * EMBED_DIM)
            for d in range(EMBED_DIM):
                vals = tbufs[slot][d, pl.ds(vg * LANES, LANES)]
                plsc.store_scatter(rbufs[slot], [bases[d] + col], vals)

    for k in range(2):
        @pl.when(k < cnt)
        def _():
            start_in(lo + k, k)

    def unit_body(i, carry):
        u = lo + i
        islot = i % 2
        for slot in range(2):
            @pl.when(islot == slot)
            def _():
                wait_in(slot)

                @pl.when(i >= 2)
                def _():
                    wait_out(slot)

                transpose_unit(slot)
                start_out(u, slot)

                @pl.when(i + 2 < cnt)
                def _():
                    start_in(u + 2, slot)
        return carry

    lax.fori_loop(0, cnt, unit_body, 0)
    for slot in range(2):
        @pl.when((cnt >= 1) & ((cnt - 1) % 2 == slot))
        def _():
            wait_out(slot)

        @pl.when((cnt >= 2) & ((cnt - 2) % 2 == slot))
        def _():
            wait_out(slot)

    @pl.when(wid == NUM_WORKERS - 1)
    def _():
        pltpu.sync_copy(tail0_hbm, tailbuf)
        pltpu.sync_copy(tailbuf, s0_hbm.at[pl.ds(VT_FULL * 128 * EMBED_DIM,
                                                 1024)])
        pltpu.sync_copy(tail1_hbm, tailbuf)
        pltpu.sync_copy(tailbuf, s1_hbm.at[pl.ds(VT_FULL * 128 * EMBED_DIM,
                                                 1024)])


BT_UNITS = BATCH // 128
BT_PER_W = BT_UNITS // NUM_WORKERS
XIN = 128 * FIELDS
OUT_WORDS = BATCH * FIELDS * 2 * EMBED_DIM
FBLK = 4 * 8 * 128


@functools.partial(
    pl.kernel,
    out_type=jax.ShapeDtypeStruct((OUT_WORDS,), jnp.float32),
    mesh=_mesh,
    scratch_types=[
        pltpu.VMEM((XIN,), jnp.int32),
        pltpu.VMEM((XIN,), jnp.int32),
        pltpu.VMEM((128, EMBED_DIM), jnp.float32),
        pltpu.VMEM((128, EMBED_DIM), jnp.float32),
        pltpu.VMEM((128, EMBED_DIM), jnp.float32),
        pltpu.VMEM((128, EMBED_DIM), jnp.float32),
        pltpu.VMEM((FBLK,), jnp.float32),
        pltpu.VMEM((FBLK,), jnp.float32),
        pltpu.SemaphoreType.DMA,
        pltpu.SemaphoreType.DMA,
        pltpu.SemaphoreType.DMA,
        pltpu.SemaphoreType.DMA,
        pltpu.SemaphoreType.DMA,
        pltpu.SemaphoreType.DMA,
    ],
    compiler_params=pltpu.CompilerParams(use_tc_tiling_on_sc=False,
                                         needs_layout_passes=False,
                                         disable_bounds_checks=True),
)
def _stack_embed(x_hbm, t0_hbm, t1_hbm, out_hbm,
                 xin, xcols, r0a, r0b, r1a, r1b, obla, oblb,
                 g0a, g0b, g1a, g1b, oa, ob):
    wid = lax.axis_index("s") * NUM_CORES + lax.axis_index("c")
    iota = lax.iota(jnp.int32, LANES)
    colbase = [iota * FIELDS + 416 * g for g in range(8)]
    bidx = [iota + LANES * g for g in range(8)]
    r0s, r1s = (r0a, r0b), (r1a, r1b)
    obls = (obla, oblb)
    gsem0, gsem1 = (g0a, g0b), (g1a, g1b)
    osems = (oa, ob)

    def start_gather(f, slot):
        idxr = xcols.at[pl.ds(f * 128, 128)]
        pltpu.async_copy(t0_hbm.at[idxr], r0s[slot], gsem0[slot])
        pltpu.async_copy(t1_hbm.at[idxr], r1s[slot], gsem1[slot])

    def wait_gather(slot):
        pltpu.make_async_copy(t0_hbm.at[pl.ds(0, 128), :],
                              r0s[slot], gsem0[slot]).wait()
        pltpu.make_async_copy(t1_hbm.at[pl.ds(0, 128), :],
                              r1s[slot], gsem1[slot]).wait()

    def start_outs(f, bt, slot):
        for dt in range(4):
            off = ((f * 4 + dt) * BT_UNITS + bt) * 1024
            pltpu.async_copy(obls[slot].at[pl.ds(dt * 1024, 1024)],
                             out_hbm.at[pl.ds(off, 1024)], osems[slot])

    def wait_outs(slot):
        pltpu.make_async_copy(obls[slot], out_hbm.at[pl.ds(0, FBLK)],
                              osems[slot]).wait()

    def assemble(slot):
        @plsc.parallel_loop(0, EMBED_DIM, unroll=4)
        def _(d):
            dv = jnp.broadcast_to(d, (LANES,))
            for g in range(8):
                v0 = plsc.load_gather(r0s[slot], [bidx[g], dv])
                obls[slot][pl.ds(d * 128 + g * LANES, LANES)] = v0
                v1 = plsc.load_gather(r1s[slot], [bidx[g], dv])
                obls[slot][pl.ds((EMBED_DIM + d) * 128 + g * LANES,
                                 LANES)] = v1

    def bt_body(j, carry):
        bt = wid * BT_PER_W + j
        pltpu.sync_copy(x_hbm.at[pl.ds(bt * XIN, XIN)], xin)

        @plsc.parallel_loop(0, FIELDS, unroll=2)
        def _(f):
            for g in range(8):
                vals = plsc.load_gather(xin, [colbase[g] + f])
                xcols[pl.ds(f * 128 + g * LANES, LANES)] = vals

        for k in range(2):
            start_gather(k, k)

        def f_body(f, c):
            for slot in range(2):
                @pl.when(f % 2 == slot)
                def _():
                    wait_gather(slot)

                    @pl.when(f >= 2)
                    def _():
                        wait_outs(slot)

                    assemble(slot)
                    start_outs(f, bt, slot)

                    @pl.when(f + 2 < FIELDS)
                    def _():
                        start_gather(f + 2, slot)
            return c

        lax.fori_loop(0, FIELDS, f_body, 0)
        for slot in range(2):
            wait_outs(slot)
        return carry

    lax.fori_loop(0, BT_PER_W, bt_body, 0)


def kernel(x, table0, table1):
    t0t = table0.T
    t1t = table1.T
    tail0 = lax.slice(table0, (VT_FULL * 128, 0), (VOCAB, EMBED_DIM))
    tail1 = lax.slice(table1, (VT_FULL * 128, 0), (VOCAB, EMBED_DIM))
    s0, s1 = _convert(t0t, t1t, tail0.reshape(1024), tail1.reshape(1024))
    out = _stack_embed(x.reshape(TOTAL),
                       s0.reshape(VOCAB, EMBED_DIM),
                       s1.reshape(VOCAB, EMBED_DIM))
    return (out.reshape(FIELDS, 4, BT_UNITS, 8, 128)
            .transpose(2, 4, 0, 1, 3)
            .reshape(BATCH, FIELDS, 2 * EMBED_DIM))
